# baseline (speedup 1.0000x reference)
import functools

import jax
import jax.numpy as jnp
from jax import lax
from jax.experimental import pallas as pl
from jax.experimental.pallas import tpu as pltpu
from jax.experimental.pallas import tpu_sc as plsc

V = 1000000
D = 64
S = 128
LANES = 16

NC, NS = 2, 16
NW = NC * NS

CHUNK = 256
SPC = CHUNK // S
NB = 2
SP = S

NBLK = (V + S - 1) // S
NFULL = V // S
BPW = (NBLK + NW - 1) // NW


def _detile_body(tt_hbm, tail_hbm, out_hbm, bufs, trs, isems, osems):
    wid = lax.axis_index("s") * NC + lax.axis_index("c")

    iota = jnp.arange(LANES, dtype=jnp.int32)
    ones = jnp.ones((LANES,), jnp.int32)
    perms = [(iota + k) % LANES for k in range(LANES)]
    rowg = [iota + d0 * LANES for d0 in range(D // LANES)]

    def start_in(b, j):
        @pl.when(j < NFULL)
        def _():
            pltpu.async_copy(
                tt_hbm.at[pl.ds(0, D), pl.ds(pl.multiple_of(j * S, S), S)],
                bufs[b], isems[b])

        @pl.when(j == NFULL)
        def _():
            pltpu.async_copy(tail_hbm, bufs[b], isems[b])

    def wait_in(b):
        pltpu.make_async_copy(
            tt_hbm.at[pl.ds(0, D), pl.ds(0, S)], bufs[b], isems[b]).wait()

    def transpose(b, v0_lo, v0_hi):
        phalf = [p >> 1 for p in perms]
        scols = [[(perms[k] & 1) * D + rowg[d0]
                  for d0 in range(D // LANES)] for k in range(LANES)]
        start = jnp.full((LANES,), v0_lo * LANES, jnp.int32)
        starth = jnp.full((LANES,), v0_lo * (LANES // 2), jnp.int32)

        @pl.loop(v0_lo, v0_hi, init_carry=(start, starth), unroll=2)
        def v_step(v0, carry):
            vv, vvh = carry
            for k in range(LANES):
                vperm = vv + perms[k]
                srow = vvh + phalf[k]
                for d0 in range(D // LANES):
                    g = plsc.load_gather(bufs[b], [rowg[d0], vperm])
                    plsc.store_scatter(trs[b], [srow, scols[k][d0]], g)
            return vv + (ones * LANES), vvh + (ones * (LANES // 2))

    def store_out(b, j):
        @pl.when(j < NFULL)
        def _():
            pltpu.async_copy(
                trs[b], out_hbm.at[pl.ds(j * (S // 2), S // 2)], osems[b])

        @pl.when(j == NFULL)
        def _():
            pltpu.async_copy(
                trs[b].at[pl.ds(0, S // 4)],
                out_hbm.at[pl.ds(NFULL * (S // 2), S // 4)], osems[b])

    def wait_out(b, nrows):
        pltpu.make_async_copy(
            trs[b].at[pl.ds(0, nrows)],
            out_hbm.at[pl.ds(0, nrows)], osems[b]).wait()

    def do_block(b, j):
        wait_in(b)

        @pl.when(j < NFULL)
        def _():
            transpose(b, 0, S // LANES)

        @pl.when(j == NFULL)
        def _():
            transpose(b, 0, D // LANES)

        store_out(b, j)

    def prime(b, j):
        start_in(b, j)

    prime(0, wid)
    prime(1, wid + NW)

    def outer(t, _):
        j0 = wid + t * (2 * NW)
        for b in range(2):
            j = j0 + b * NW

            @pl.when(j < NBLK)
            def _():
                do_block(b, j)
                j2 = j + 2 * NW

                @pl.when(j2 < NBLK)
                def _():
                    wait_out(b, S // 2)
                    prime(b, j2)
        return ()

    lax.fori_loop(0, (BPW + 1) // 2, outer, ())

    nblocks_w = (NBLK - wid + NW - 1) // NW
    last = wid + (nblocks_w - 1) * NW
    b_last = (nblocks_w - 1) % 2

    def drain_last(b, j):
        @pl.when(j < NFULL)
        def _():
            wait_out(b, S // 2)

        @pl.when(j == NFULL)
        def _():
            wait_out(b, D // 2)

    @pl.when(b_last == 0)
    def _():
        drain_last(0, last)
        wait_out(1, S // 2)

    @pl.when(b_last == 1)
    def _():
        drain_last(1, last)
        wait_out(0, S // 2)


def _embed_body(x_hbm, tok_hbm, pos_hbm, out_hbm, pos_v, idxs, rows, trs,
                gsems, ssems):
    n_batch = out_hbm.shape[0]
    per_w = n_batch // NW
    n_chunks = per_w // SPC
    wid = lax.axis_index("s") * NC + lax.axis_index("c")
    tok_base = wid * per_w * S
    seq_base = wid * per_w

    pltpu.sync_copy(pos_hbm, pos_v)

    ones = jnp.ones((LANES,), jnp.int32)
    zeros = jnp.zeros((LANES,), jnp.int32)

    def start_gather(b, g):
        pltpu.sync_copy(x_hbm.at[pl.ds(tok_base + g * CHUNK, CHUNK)], idxs[b])
        pltpu.async_copy(tok_hbm.at[idxs[b]], rows[b], gsems[b])

    def stage(b, g):
        pltpu.make_async_copy(tok_hbm.at[idxs[b]], rows[b], gsems[b]).wait()

        @pl.when(g >= NB)
        def _():
            for q in range(SPC):
                pltpu.make_async_copy(
                    trs[b].at[pl.ds(q * D, D), pl.ds(0, S)],
                    out_hbm.at[seq_base], ssems[b]).wait()

        row_idx = [[jnp.arange(LANES, dtype=jnp.int32) + q * D + d0 * LANES
                    for d0 in range(D // LANES)] for q in range(SPC)]

        @pl.loop(0, S, init_carry=zeros, unroll=8)
        def s_step(s, ss):
            rp = pos_v.at[s]
            for q in range(SPC):
                rq = rows[b].at[q * S + s]
                for d0 in range(D // LANES):
                    pv = rp[pl.ds(d0 * LANES, LANES)]
                    v = rq[pl.ds(d0 * LANES, LANES)]
                    plsc.store_scatter(trs[b], [row_idx[q][d0], ss], v + pv)
            return ss + ones

        g2 = g + NB

        @pl.when(g2 < n_chunks)
        def _():
            start_gather(b, g2)

        for q in range(SPC):
            pltpu.async_copy(
                trs[b].at[pl.ds(q * D, D), pl.ds(0, S)],
                out_hbm.at[seq_base + g * SPC + q], ssems[b])

    for b in range(NB):
        start_gather(b, b)

    def outer(gg, _):
        for b in range(NB):
            stage(b, gg * NB + b)
        return ()

    lax.fori_loop(0, n_chunks // NB, outer, ())
    for b in range(NB):
        for q in range(SPC):
            pltpu.make_async_copy(
                trs[b].at[pl.ds(q * D, D), pl.ds(0, S)],
                out_hbm.at[seq_base], ssems[b]).wait()


@jax.jit
def _run(x, token_table, pos_table):
    mesh = plsc.VectorSubcoreMesh(
        core_axis_name="c", subcore_axis_name="s",
        num_cores=NC, num_subcores=NS,
    )

    detile = pl.kernel(
        _detile_body,
        out_type=jax.ShapeDtypeStruct((V // 2, S), jnp.float32),
        mesh=mesh,
        scratch_types=[
            [pltpu.VMEM((D, S), jnp.float32) for _ in range(2)],
            [pltpu.VMEM((D, S), jnp.float32) for _ in range(2)],
            [pltpu.SemaphoreType.DMA for _ in range(2)],
            [pltpu.SemaphoreType.DMA for _ in range(2)],
        ],
        compiler_params=pltpu.CompilerParams(
            use_tc_tiling_on_sc=True, needs_layout_passes=False),
    )
    tail = jnp.pad(token_table[NFULL * S:].T, ((0, 0), (0, S - (V - NFULL * S))))
    tabc = detile(token_table.T, tail)
    tok_lin = tabc.reshape(V, D)

    n_tokens = x.shape[0] * x.shape[1]
    n_batch = x.shape[0] * x.shape[1] // S
    embed = pl.kernel(
        _embed_body,
        out_type=jax.ShapeDtypeStruct((n_batch, D, S), jnp.float32),
        mesh=mesh,
        scratch_types=[
            pltpu.VMEM((S, D), jnp.float32),
            [pltpu.VMEM((CHUNK,), jnp.int32) for _ in range(NB)],
            [pltpu.VMEM((CHUNK, D), jnp.float32) for _ in range(NB)],
            [pltpu.VMEM((SPC * D, SP), jnp.float32) for _ in range(NB)],
            [pltpu.SemaphoreType.DMA for _ in range(NB)],
            [pltpu.SemaphoreType.DMA for _ in range(NB)],
        ],
        compiler_params=pltpu.CompilerParams(
            use_tc_tiling_on_sc=False, needs_layout_passes=False),
    )
    out_t = embed(x.reshape(n_tokens), tok_lin, pos_table)
    return jnp.transpose(out_t, (0, 2, 1))


def kernel(x, token_table, pos_table):
    return _run(x, token_table, pos_table)


# speedup vs baseline: 1.3911x; 1.3911x over previous
import functools

import jax
import jax.numpy as jnp
from jax import lax
from jax.experimental import pallas as pl
from jax.experimental.pallas import tpu as pltpu
from jax.experimental.pallas import tpu_sc as plsc

V = 1000000
D = 64
S = 128
LANES = 16

NC, NS = 2, 16
NW = NC * NS

CHUNK = 256
SPC = CHUNK // S
NB = 2
SP = S + 1

NBLK = (V + S - 1) // S
NFULL = V // S
BPW = (NBLK + NW - 1) // NW


def _detile_body(tt_hbm, tail_hbm, out_hbm, bufs, trs, isems, osems):
    wid = lax.axis_index("s") * NC + lax.axis_index("c")

    iota = jnp.arange(LANES, dtype=jnp.int32)
    ones = jnp.ones((LANES,), jnp.int32)
    perms = [(iota + k) % LANES for k in range(LANES)]
    rowg = [iota + d0 * LANES for d0 in range(D // LANES)]

    def start_in(b, j):
        @pl.when(j < NFULL)
        def _():
            pltpu.async_copy(
                tt_hbm.at[pl.ds(0, D), pl.ds(pl.multiple_of(j * S, S), S)],
                bufs[b], isems[b])

        @pl.when(j == NFULL)
        def _():
            pltpu.async_copy(tail_hbm, bufs[b], isems[b])

    def wait_in(b):
        pltpu.make_async_copy(
            tt_hbm.at[pl.ds(0, D), pl.ds(0, S)], bufs[b], isems[b]).wait()

    def transpose(b, v0_lo, v0_hi):
        phalf = [p >> 1 for p in perms]
        scols = [[(perms[k] & 1) * D + rowg[d0]
                  for d0 in range(D // LANES)] for k in range(LANES)]
        start = jnp.full((LANES,), v0_lo * LANES, jnp.int32)
        starth = jnp.full((LANES,), v0_lo * (LANES // 2), jnp.int32)

        @pl.loop(v0_lo, v0_hi, init_carry=(start, starth), unroll=2)
        def v_step(v0, carry):
            vv, vvh = carry
            for k in range(LANES):
                vperm = vv + perms[k]
                srow = vvh + phalf[k]
                for d0 in range(D // LANES):
                    g = plsc.load_gather(bufs[b], [rowg[d0], vperm])
                    plsc.store_scatter(trs[b], [srow, scols[k][d0]], g)
            return vv + (ones * LANES), vvh + (ones * (LANES // 2))

    def store_out(b, j):
        @pl.when(j < NFULL)
        def _():
            pltpu.async_copy(
                trs[b], out_hbm.at[pl.ds(j * (S // 2), S // 2)], osems[b])

        @pl.when(j == NFULL)
        def _():
            pltpu.async_copy(
                trs[b].at[pl.ds(0, S // 4)],
                out_hbm.at[pl.ds(NFULL * (S // 2), S // 4)], osems[b])

    def wait_out(b, nrows):
        pltpu.make_async_copy(
            trs[b].at[pl.ds(0, nrows)],
            out_hbm.at[pl.ds(0, nrows)], osems[b]).wait()

    def do_block(b, j):
        wait_in(b)

        @pl.when(j < NFULL)
        def _():
            transpose(b, 0, S // LANES)

        @pl.when(j == NFULL)
        def _():
            transpose(b, 0, D // LANES)

        store_out(b, j)

    def prime(b, j):
        start_in(b, j)

    prime(0, wid)
    prime(1, wid + NW)

    def outer(t, _):
        j0 = wid + t * (2 * NW)
        for b in range(2):
            j = j0 + b * NW

            @pl.when(j < NBLK)
            def _():
                do_block(b, j)
                j2 = j + 2 * NW

                @pl.when(j2 < NBLK)
                def _():
                    wait_out(b, S // 2)
                    prime(b, j2)
        return ()

    lax.fori_loop(0, (BPW + 1) // 2, outer, ())

    nblocks_w = (NBLK - wid + NW - 1) // NW
    last = wid + (nblocks_w - 1) * NW
    b_last = (nblocks_w - 1) % 2

    def drain_last(b, j):
        @pl.when(j < NFULL)
        def _():
            wait_out(b, S // 2)

        @pl.when(j == NFULL)
        def _():
            wait_out(b, D // 2)

    @pl.when(b_last == 0)
    def _():
        drain_last(0, last)
        wait_out(1, S // 2)

    @pl.when(b_last == 1)
    def _():
        drain_last(1, last)
        wait_out(0, S // 2)


def _embed_body(x_hbm, tok_hbm, pos_hbm, out_hbm, pos_v, idxs, rows, trs,
                gsems, ssems):
    n_batch = out_hbm.shape[0]
    per_w = n_batch // NW
    n_chunks = per_w // SPC
    wid = lax.axis_index("s") * NC + lax.axis_index("c")
    tok_base = wid * per_w * S
    seq_base = wid * per_w

    pltpu.sync_copy(pos_hbm, pos_v)

    ones = jnp.ones((LANES,), jnp.int32)
    zeros = jnp.zeros((LANES,), jnp.int32)

    def start_gather(b, g):
        pltpu.sync_copy(x_hbm.at[pl.ds(tok_base + g * CHUNK, CHUNK)], idxs[b])
        pltpu.async_copy(tok_hbm.at[idxs[b]], rows[b], gsems[b])

    def stage(b, g):
        pltpu.make_async_copy(tok_hbm.at[idxs[b]], rows[b], gsems[b]).wait()

        @pl.when(g >= NB)
        def _():
            for q in range(SPC):
                pltpu.make_async_copy(
                    trs[b].at[pl.ds(q * D, D), pl.ds(0, S)],
                    out_hbm.at[seq_base], ssems[b]).wait()

        row_idx = [[jnp.arange(LANES, dtype=jnp.int32) + q * D + d0 * LANES
                    for d0 in range(D // LANES)] for q in range(SPC)]

        @pl.loop(0, S, init_carry=zeros, unroll=4)
        def s_step(s, ss):
            for d0 in range(D // LANES):
                pv = pos_v[s, pl.ds(d0 * LANES, LANES)]
                for q in range(SPC):
                    v = rows[b][q * S + s, pl.ds(d0 * LANES, LANES)]
                    plsc.store_scatter(trs[b], [row_idx[q][d0], ss], v + pv)
            return ss + ones

        g2 = g + NB

        @pl.when(g2 < n_chunks)
        def _():
            start_gather(b, g2)

        for q in range(SPC):
            pltpu.async_copy(
                trs[b].at[pl.ds(q * D, D), pl.ds(0, S)],
                out_hbm.at[seq_base + g * SPC + q], ssems[b])

    for b in range(NB):
        start_gather(b, b)

    def outer(gg, _):
        for b in range(NB):
            stage(b, gg * NB + b)
        return ()

    lax.fori_loop(0, n_chunks // NB, outer, ())
    for b in range(NB):
        for q in range(SPC):
            pltpu.make_async_copy(
                trs[b].at[pl.ds(q * D, D), pl.ds(0, S)],
                out_hbm.at[seq_base], ssems[b]).wait()


@jax.jit
def _run(x, token_table, pos_table):
    mesh = plsc.VectorSubcoreMesh(
        core_axis_name="c", subcore_axis_name="s",
        num_cores=NC, num_subcores=NS,
    )

    detile = pl.kernel(
        _detile_body,
        out_type=jax.ShapeDtypeStruct((V // 2, S), jnp.float32),
        mesh=mesh,
        scratch_types=[
            [pltpu.VMEM((D, S), jnp.float32) for _ in range(2)],
            [pltpu.VMEM((D, S), jnp.float32) for _ in range(2)],
            [pltpu.SemaphoreType.DMA for _ in range(2)],
            [pltpu.SemaphoreType.DMA for _ in range(2)],
        ],
        compiler_params=pltpu.CompilerParams(
            use_tc_tiling_on_sc=True, needs_layout_passes=False),
    )
    tail = jnp.pad(token_table[NFULL * S:].T, ((0, 0), (0, S - (V - NFULL * S))))
    tabc = detile(token_table.T, tail)
    tok_lin = tabc.reshape(V, D)

    n_tokens = x.shape[0] * x.shape[1]
    n_batch = x.shape[0] * x.shape[1] // S
    embed = pl.kernel(
        _embed_body,
        out_type=jax.ShapeDtypeStruct((n_batch, D, S), jnp.float32),
        mesh=mesh,
        scratch_types=[
            pltpu.VMEM((S, D), jnp.float32),
            [pltpu.VMEM((CHUNK,), jnp.int32) for _ in range(NB)],
            [pltpu.VMEM((CHUNK, D), jnp.float32) for _ in range(NB)],
            [pltpu.VMEM((SPC * D, SP), jnp.float32) for _ in range(NB)],
            [pltpu.SemaphoreType.DMA for _ in range(NB)],
            [pltpu.SemaphoreType.DMA for _ in range(NB)],
        ],
        compiler_params=pltpu.CompilerParams(
            use_tc_tiling_on_sc=False, needs_layout_passes=False),
    )
    out_t = embed(x.reshape(n_tokens), tok_lin, pos_table)
    return jnp.transpose(out_t, (0, 2, 1))


def kernel(x, token_table, pos_table):
    return _run(x, token_table, pos_table)
